# baseline (speedup 1.0000x reference)
import jax
import jax.numpy as jnp
from jax.experimental import pallas as pl
from jax.experimental.pallas import tpu as pltpu

IN_DIM = 1024
OUT_DIM = 1024
K = 2048
B_TOK = 8192
TM = 1024


def _fused_body(x_ref, A_ref, B_ref, out_ref, inner_ref, C_s):
    i = pl.program_id(0)

    @pl.when(i == 0)
    def _prep():
        a = A_ref[...]
        inv = jax.lax.rsqrt(jnp.sum(a * a, axis=0, keepdims=True))
        an = (a * inv).astype(jnp.bfloat16)
        C_s[:, :K] = an
        C_s[:, K:] = jnp.dot(
            an, B_ref[...].astype(jnp.bfloat16),
            preferred_element_type=jnp.float32).astype(jnp.bfloat16)

    xb = x_ref[...]
    inner_ref[...] = jnp.dot(xb, C_s[:, :K], preferred_element_type=jnp.float32)
    out_ref[...] = jnp.dot(xb, C_s[:, K:], preferred_element_type=jnp.float32)


def kernel(x, A, Bm):
    xb = x.astype(jnp.bfloat16)
    n_tiles = B_TOK // TM
    out, inner = pl.pallas_call(
        _fused_body,
        grid=(n_tiles,),
        in_specs=[
            pl.BlockSpec((TM, IN_DIM), lambda i: (i, 0)),
            pl.BlockSpec((IN_DIM, K), lambda i: (0, 0)),
            pl.BlockSpec((K, OUT_DIM), lambda i: (0, 0)),
        ],
        out_specs=[
            pl.BlockSpec((TM, OUT_DIM), lambda i: (i, 0)),
            pl.BlockSpec((TM, K), lambda i: (i, 0)),
        ],
        out_shape=[
            jax.ShapeDtypeStruct((B_TOK, OUT_DIM), jnp.float32),
            jax.ShapeDtypeStruct((B_TOK, K), jnp.float32),
        ],
        scratch_shapes=[
            pltpu.VMEM((IN_DIM, K + OUT_DIM), jnp.bfloat16),
        ],
        compiler_params=pltpu.CompilerParams(
            dimension_semantics=("arbitrary",),
            vmem_limit_bytes=62 * 1024 * 1024,
            allow_input_fusion=[True, False, False],
        ),
    )(xb, A, Bm)
    return (out, inner)


# speedup vs baseline: 1.2436x; 1.2436x over previous
import jax
import jax.numpy as jnp
from jax.experimental import pallas as pl
from jax.experimental.pallas import tpu as pltpu

IN_DIM = 1024
OUT_DIM = 1024
K = 2048
B_TOK = 8192
TM = 1024


def _fused_body(x_ref, A_ref, B_ref, out_ref, inner_ref, C_s, xb_s):
    i = pl.program_id(0)

    @pl.when(i == 0)
    def _prep():
        a = A_ref[...]
        inv = jax.lax.rsqrt(jnp.sum(a * a, axis=0, keepdims=True))
        an = (a * inv).astype(jnp.bfloat16)
        C_s[:, :K] = an
        C_s[:, K:] = jnp.dot(
            an, B_ref[...].astype(jnp.bfloat16),
            preferred_element_type=jnp.float32).astype(jnp.bfloat16)

    xb_s[...] = x_ref[...].astype(jnp.bfloat16)
    xb = xb_s[...]
    inner_ref[...] = jnp.dot(xb, C_s[:, :K], preferred_element_type=jnp.float32)
    out_ref[...] = jnp.dot(xb, C_s[:, K:], preferred_element_type=jnp.float32)


def kernel(x, A, Bm):
    n_tiles = B_TOK // TM
    out, inner = pl.pallas_call(
        _fused_body,
        grid=(n_tiles,),
        in_specs=[
            pl.BlockSpec((TM, IN_DIM), lambda i: (i, 0)),
            pl.BlockSpec((IN_DIM, K), lambda i: (0, 0)),
            pl.BlockSpec((K, OUT_DIM), lambda i: (0, 0)),
        ],
        out_specs=[
            pl.BlockSpec((TM, OUT_DIM), lambda i: (i, 0)),
            pl.BlockSpec((TM, K), lambda i: (i, 0)),
        ],
        out_shape=[
            jax.ShapeDtypeStruct((B_TOK, OUT_DIM), jnp.float32),
            jax.ShapeDtypeStruct((B_TOK, K), jnp.float32),
        ],
        scratch_shapes=[
            pltpu.VMEM((IN_DIM, K + OUT_DIM), jnp.bfloat16),
            pltpu.VMEM((TM, IN_DIM), jnp.bfloat16),
        ],
        compiler_params=pltpu.CompilerParams(
            dimension_semantics=("arbitrary",),
            vmem_limit_bytes=62 * 1024 * 1024,
        ),
    )(x, A, Bm)
    return (out, inner)
